# baseline (speedup 1.0000x reference)
import jax
import jax.numpy as jnp
from jax import lax
from jax.experimental import pallas as pl
from jax.experimental.pallas import tpu as pltpu
from jax.experimental.pallas import tpu_sc as plsc

B, S, C = 4, 8192, 768
K = 2
L = 16

TC_ROWS = 4096
SC_ROWS = S - TC_ROWS

SLABS = 8
ROWS_W = SC_ROWS // SLABS
RCHUNK = 64
NCHUNK = ROWS_W // RCHUNK
CG = C // L
CPB = C // SLABS
RUNROLL = 4

BS = 512

_NEG = float("-inf")


def _sc_body(x_hbm, out1_hbm, out2_hbm, buf0, buf1, state, shared, mbuf,
             obuf1, obuf2, sem0, sem1):
    cid = lax.axis_index("c")
    sid = lax.axis_index("s")
    grp = sid // SLABS
    slab = sid % SLABS
    b = cid * 2 + grp
    r0 = b * S + TC_ROWS + slab * ROWS_W

    bufs = (buf0, buf1)
    sems = (sem0, sem1)
    cps = [None, None]
    cps[0] = pltpu.async_copy(
        x_hbm.at[pl.ds(pl.multiple_of(r0, RCHUNK), RCHUNK), :], buf0, sem0)
    for i in range(NCHUNK):
        if i + 1 < NCHUNK:
            j = (i + 1) % 2
            cps[j] = pltpu.async_copy(
                x_hbm.at[pl.ds(pl.multiple_of(r0 + (i + 1) * RCHUNK, RCHUNK), RCHUNK), :],
                bufs[j], sems[j])
        cps[i % 2].wait()
        buf = bufs[i % 2]
        first = i == 0

        def cg_body(cg, _, buf=buf, first=first):
            col = cg * L
            if first:
                m1 = jnp.full((L,), _NEG, jnp.float32)
                m2 = jnp.full((L,), _NEG, jnp.float32)
            else:
                m1 = state[pl.ds(col, L)]
                m2 = state[pl.ds(C + col, L)]

            def row_body(r, carry):
                m1, m2 = carry
                for u in range(RUNROLL):
                    v = buf[r * RUNROLL + u, pl.ds(col, L)]
                    m2 = jnp.maximum(m2, jnp.minimum(m1, v))
                    m1 = jnp.maximum(m1, v)
                return m1, m2

            m1, m2 = lax.fori_loop(0, RCHUNK // RUNROLL, row_body, (m1, m2))
            state[pl.ds(col, L)] = m1
            state[pl.ds(C + col, L)] = m2
            return 0

        lax.fori_loop(0, CG, cg_body, 0)

    pltpu.sync_copy(state, shared.at[pl.ds(pl.multiple_of(sid * 2 * C, 8), 2 * C)])
    plsc.subcore_barrier()
    pltpu.sync_copy(
        shared.at[pl.ds(pl.multiple_of(grp * SLABS * 2 * C, 8), SLABS * 2 * C)], mbuf)

    col0 = slab * CPB

    def mg_body(j, _):
        cc = col0 + j * L
        m1 = jnp.full((L,), _NEG, jnp.float32)
        m2 = jnp.full((L,), _NEG, jnp.float32)
        for t in range(SLABS):
            a1 = mbuf[pl.ds(t * 2 * C + cc, L)]
            a2 = mbuf[pl.ds(t * 2 * C + C + cc, L)]
            m2 = jnp.maximum(jnp.maximum(m2, a2), jnp.minimum(m1, a1))
            m1 = jnp.maximum(m1, a1)
        obuf1[pl.ds(j * L, L)] = m1
        obuf2[pl.ds(j * L, L)] = m2
        return 0

    lax.fori_loop(0, CPB // L, mg_body, 0)
    o_off = pl.multiple_of(b * C + col0, 32)
    pltpu.sync_copy(obuf1, out1_hbm.at[pl.ds(o_off, CPB)])
    pltpu.sync_copy(obuf2, out2_hbm.at[pl.ds(o_off, CPB)])


def _tc_body(x_ref, o1_ref, o2_ref, m1_s, m2_s):
    t = pl.program_id(1)
    nt = pl.num_programs(1)

    @pl.when(t == 0)
    def _():
        m1_s[...] = jnp.full(m1_s.shape, _NEG, jnp.float32)
        m2_s[...] = jnp.full(m2_s.shape, _NEG, jnp.float32)

    v = x_ref[0]
    a, bb = v[:BS // 2], v[BS // 2:]
    m1 = jnp.maximum(a, bb)
    m2 = jnp.minimum(a, bb)
    h = BS // 4
    while h >= 8:
        a1, b1 = m1[:h], m1[h:]
        a2, b2 = m2[:h], m2[h:]
        m2 = jnp.maximum(jnp.maximum(a2, b2), jnp.minimum(a1, b1))
        m1 = jnp.maximum(a1, b1)
        h //= 2
    s1, s2 = m1_s[...], m2_s[...]
    m1_s[...] = jnp.maximum(s1, m1)
    m2_s[...] = jnp.maximum(jnp.maximum(s2, m2), jnp.minimum(s1, m1))

    @pl.when(t == nt - 1)
    def _():
        m1, m2 = m1_s[...], m2_s[...]
        for h in (4, 2, 1):
            a1, b1 = m1[:h], m1[h:2 * h]
            a2, b2 = m2[:h], m2[h:2 * h]
            m2 = jnp.maximum(jnp.maximum(a2, b2), jnp.minimum(a1, b1))
            m1 = jnp.maximum(a1, b1)
        bi = pl.program_id(0)
        o1_ref[pl.ds(bi, 1), :] = m1
        o2_ref[pl.ds(bi, 1), :] = m2


def _merge_body(s1_ref, s2_ref, t1_ref, t2_ref, o1_ref, o2_ref):
    a1, a2 = s1_ref[...], s2_ref[...]
    b1, b2 = t1_ref[...], t2_ref[...]
    o2_ref[...] = jnp.maximum(jnp.maximum(a2, b2), jnp.minimum(a1, b1))
    o1_ref[...] = jnp.maximum(a1, b1)


def kernel(inputs):
    x2d = inputs.reshape(B * S, C)
    mesh = plsc.VectorSubcoreMesh(
        core_axis_name="c", subcore_axis_name="s", num_cores=2, num_subcores=16)
    sc_k = pl.kernel(
        _sc_body,
        out_type=(jax.ShapeDtypeStruct((B * C,), jnp.float32),
                  jax.ShapeDtypeStruct((B * C,), jnp.float32)),
        mesh=mesh,
        scratch_types=[
            pltpu.VMEM((RCHUNK, C), jnp.float32),
            pltpu.VMEM((RCHUNK, C), jnp.float32),
            pltpu.VMEM((2 * C,), jnp.float32),
            pltpu.VMEM_SHARED((16 * 2 * C,), jnp.float32),
            pltpu.VMEM((SLABS * 2 * C,), jnp.float32),
            pltpu.VMEM((CPB,), jnp.float32),
            pltpu.VMEM((CPB,), jnp.float32),
            pltpu.SemaphoreType.DMA,
            pltpu.SemaphoreType.DMA,
        ],
    )
    sc1, sc2 = sc_k(x2d)

    tc1, tc2 = pl.pallas_call(
        _tc_body,
        grid=(B, TC_ROWS // BS),
        in_specs=[pl.BlockSpec((1, BS, C), lambda b, t: (b, t, 0))],
        out_specs=(pl.BlockSpec((B, C), lambda b, t: (0, 0)),
                   pl.BlockSpec((B, C), lambda b, t: (0, 0))),
        out_shape=(jax.ShapeDtypeStruct((B, C), jnp.float32),
                   jax.ShapeDtypeStruct((B, C), jnp.float32)),
        scratch_shapes=[pltpu.VMEM((8, C), jnp.float32),
                        pltpu.VMEM((8, C), jnp.float32)],
        compiler_params=pltpu.CompilerParams(
            dimension_semantics=("arbitrary", "arbitrary")),
    )(inputs)

    o1, o2 = pl.pallas_call(
        _merge_body,
        out_shape=(jax.ShapeDtypeStruct((B, C), jnp.float32),
                   jax.ShapeDtypeStruct((B, C), jnp.float32)),
    )(sc1.reshape(B, C), sc2.reshape(B, C), tc1, tc2)

    return jnp.stack([o1, o2], axis=-1).reshape(B, C * K)


# speedup vs baseline: 1.0247x; 1.0247x over previous
import jax
import jax.numpy as jnp
from jax import lax
from jax.experimental import pallas as pl
from jax.experimental.pallas import tpu as pltpu
from jax.experimental.pallas import tpu_sc as plsc

B, S, C = 4, 8192, 768
K = 2
L = 16

TC_ROWS = 4096
SC_ROWS = S - TC_ROWS

SLABS = 8
ROWS_W = SC_ROWS // SLABS
RCHUNK = 64
NCHUNK = ROWS_W // RCHUNK
CG = C // L
CPB = C // SLABS
RUNROLL = 4

BS = 512

_NEG = float("-inf")


def _sc_body(x_hbm, out1_hbm, out2_hbm, buf0, buf1, state, shared, mbuf,
             obuf1, obuf2, sem0, sem1):
    cid = lax.axis_index("c")
    sid = lax.axis_index("s")
    grp = sid // SLABS
    slab = sid % SLABS
    b = cid * 2 + grp
    r0 = b * S + TC_ROWS + slab * ROWS_W

    def init_body(jj, _):
        state[pl.ds(jj * L, L)] = jnp.full((L,), _NEG, jnp.float32)
        return 0

    lax.fori_loop(0, 2 * C // L, init_body, 0)

    bufs = (buf0, buf1)
    sems = (sem0, sem1)
    pltpu.async_copy(
        x_hbm.at[pl.ds(pl.multiple_of(r0, RCHUNK), RCHUNK), :], buf0, sem0)

    def chunk_pair(i, _):
        for par in range(2):
            j = i * 2 + par
            nbuf, nsem = bufs[(par + 1) % 2], sems[(par + 1) % 2]

            @pl.when(j + 1 < NCHUNK)
            def _():
                pltpu.async_copy(
                    x_hbm.at[pl.ds(pl.multiple_of(r0 + (j + 1) * RCHUNK, RCHUNK),
                                   RCHUNK), :],
                    nbuf, nsem)

            pltpu.make_async_copy(
                x_hbm.at[pl.ds(0, RCHUNK), :], bufs[par], sems[par]).wait()
            buf = bufs[par]

            def cg_body(cg, _, buf=buf):
                col = cg * L
                m1 = state[pl.ds(col, L)]
                m2 = state[pl.ds(C + col, L)]

                def row_body(r, carry):
                    m1, m2 = carry
                    for u in range(RUNROLL):
                        v = buf[r * RUNROLL + u, pl.ds(col, L)]
                        m2 = jnp.maximum(m2, jnp.minimum(m1, v))
                        m1 = jnp.maximum(m1, v)
                    return m1, m2

                m1, m2 = lax.fori_loop(0, RCHUNK // RUNROLL, row_body, (m1, m2))
                state[pl.ds(col, L)] = m1
                state[pl.ds(C + col, L)] = m2
                return 0

            lax.fori_loop(0, CG, cg_body, 0)
        return 0

    lax.fori_loop(0, NCHUNK // 2, chunk_pair, 0)

    pltpu.sync_copy(state, shared.at[pl.ds(pl.multiple_of(sid * 2 * C, 8), 2 * C)])
    plsc.subcore_barrier()
    pltpu.sync_copy(
        shared.at[pl.ds(pl.multiple_of(grp * SLABS * 2 * C, 8), SLABS * 2 * C)], mbuf)

    col0 = slab * CPB

    def mg_body(j, _):
        cc = col0 + j * L
        m1 = jnp.full((L,), _NEG, jnp.float32)
        m2 = jnp.full((L,), _NEG, jnp.float32)
        for t in range(SLABS):
            a1 = mbuf[pl.ds(t * 2 * C + cc, L)]
            a2 = mbuf[pl.ds(t * 2 * C + C + cc, L)]
            m2 = jnp.maximum(jnp.maximum(m2, a2), jnp.minimum(m1, a1))
            m1 = jnp.maximum(m1, a1)
        obuf1[pl.ds(j * L, L)] = m1
        obuf2[pl.ds(j * L, L)] = m2
        return 0

    lax.fori_loop(0, CPB // L, mg_body, 0)
    o_off = pl.multiple_of(b * C + col0, 32)
    pltpu.sync_copy(obuf1, out1_hbm.at[pl.ds(o_off, CPB)])
    pltpu.sync_copy(obuf2, out2_hbm.at[pl.ds(o_off, CPB)])


def _tc_body(x_ref, o1_ref, o2_ref, m1_s, m2_s):
    t = pl.program_id(1)
    nt = pl.num_programs(1)

    @pl.when(t == 0)
    def _():
        m1_s[...] = jnp.full(m1_s.shape, _NEG, jnp.float32)
        m2_s[...] = jnp.full(m2_s.shape, _NEG, jnp.float32)

    v = x_ref[0]
    a, bb = v[:BS // 2], v[BS // 2:]
    m1 = jnp.maximum(a, bb)
    m2 = jnp.minimum(a, bb)
    h = BS // 4
    while h >= 8:
        a1, b1 = m1[:h], m1[h:]
        a2, b2 = m2[:h], m2[h:]
        m2 = jnp.maximum(jnp.maximum(a2, b2), jnp.minimum(a1, b1))
        m1 = jnp.maximum(a1, b1)
        h //= 2
    s1, s2 = m1_s[...], m2_s[...]
    m1_s[...] = jnp.maximum(s1, m1)
    m2_s[...] = jnp.maximum(jnp.maximum(s2, m2), jnp.minimum(s1, m1))

    @pl.when(t == nt - 1)
    def _():
        o1_ref[...] = m1_s[...]
        o2_ref[...] = m2_s[...]


def _merge_body(s1_ref, s2_ref, t1_ref, t2_ref, o1_ref, o2_ref):
    for b in range(B):
        m1 = t1_ref[b * 8:(b + 1) * 8, :]
        m2 = t2_ref[b * 8:(b + 1) * 8, :]
        for h in (4, 2, 1):
            a1, b1 = m1[:h], m1[h:2 * h]
            a2, b2 = m2[:h], m2[h:2 * h]
            m2 = jnp.maximum(jnp.maximum(a2, b2), jnp.minimum(a1, b1))
            m1 = jnp.maximum(a1, b1)
        b1, b2 = m1.reshape(C), m2.reshape(C)
        a1 = s1_ref[pl.ds(b * C, C)]
        a2 = s2_ref[pl.ds(b * C, C)]
        o2_ref[pl.ds(b * C, C)] = jnp.maximum(
            jnp.maximum(a2, b2), jnp.minimum(a1, b1))
        o1_ref[pl.ds(b * C, C)] = jnp.maximum(a1, b1)


def kernel(inputs):
    x2d = inputs.reshape(B * S, C)
    mesh = plsc.VectorSubcoreMesh(
        core_axis_name="c", subcore_axis_name="s", num_cores=2, num_subcores=16)
    sc_k = pl.kernel(
        _sc_body,
        out_type=(jax.ShapeDtypeStruct((B * C,), jnp.float32),
                  jax.ShapeDtypeStruct((B * C,), jnp.float32)),
        mesh=mesh,
        scratch_types=[
            pltpu.VMEM((RCHUNK, C), jnp.float32),
            pltpu.VMEM((RCHUNK, C), jnp.float32),
            pltpu.VMEM((2 * C,), jnp.float32),
            pltpu.VMEM_SHARED((16 * 2 * C,), jnp.float32),
            pltpu.VMEM((SLABS * 2 * C,), jnp.float32),
            pltpu.VMEM((CPB,), jnp.float32),
            pltpu.VMEM((CPB,), jnp.float32),
            pltpu.SemaphoreType.DMA,
            pltpu.SemaphoreType.DMA,
        ],
    )
    sc1, sc2 = sc_k(x2d)

    tc1, tc2 = pl.pallas_call(
        _tc_body,
        grid=(B, TC_ROWS // BS),
        in_specs=[pl.BlockSpec((1, BS, C), lambda b, t: (b, t, 0))],
        out_specs=(pl.BlockSpec((8, C), lambda b, t: (b, 0)),
                   pl.BlockSpec((8, C), lambda b, t: (b, 0))),
        out_shape=(jax.ShapeDtypeStruct((B * 8, C), jnp.float32),
                   jax.ShapeDtypeStruct((B * 8, C), jnp.float32)),
        scratch_shapes=[pltpu.VMEM((8, C), jnp.float32),
                        pltpu.VMEM((8, C), jnp.float32)],
        compiler_params=pltpu.CompilerParams(
            dimension_semantics=("arbitrary", "arbitrary")),
    )(inputs)

    o1, o2 = pl.pallas_call(
        _merge_body,
        out_shape=(jax.ShapeDtypeStruct((B * C,), jnp.float32),
                   jax.ShapeDtypeStruct((B * C,), jnp.float32)),
    )(sc1, sc2, tc1, tc2)

    return jnp.stack([o1, o2], axis=-1).reshape(B, C * K)


# speedup vs baseline: 1.0845x; 1.0583x over previous
import jax
import jax.numpy as jnp
from jax import lax
from jax.experimental import pallas as pl
from jax.experimental.pallas import tpu as pltpu
from jax.experimental.pallas import tpu_sc as plsc

B, S, C = 4, 8192, 768
K = 2
L = 16

TC_ROWS = 4096
SC_ROWS = S - TC_ROWS

SLABS = 8
ROWS_W = SC_ROWS // SLABS
RCHUNK = 64
NCHUNK = ROWS_W // RCHUNK
CG = C // L
CPB = C // SLABS
RUNROLL = 8

BS = 512

_NEG = float("-inf")


def _sc_body(x_hbm, out1_hbm, out2_hbm, buf0, buf1, state, shared, mbuf,
             obuf1, obuf2, sem0, sem1):
    cid = lax.axis_index("c")
    sid = lax.axis_index("s")
    grp = sid // SLABS
    slab = sid % SLABS
    b = cid * 2 + grp
    r0 = b * S + TC_ROWS + slab * ROWS_W

    def init_body(jj, _):
        state[pl.ds(jj * L, L)] = jnp.full((L,), _NEG, jnp.float32)
        return 0

    lax.fori_loop(0, 2 * C // L, init_body, 0)

    bufs = (buf0, buf1)
    sems = (sem0, sem1)
    pltpu.async_copy(
        x_hbm.at[pl.ds(pl.multiple_of(r0, RCHUNK), RCHUNK), :], buf0, sem0)

    def chunk_pair(i, _):
        for par in range(2):
            j = i * 2 + par
            nbuf, nsem = bufs[(par + 1) % 2], sems[(par + 1) % 2]

            @pl.when(j + 1 < NCHUNK)
            def _():
                pltpu.async_copy(
                    x_hbm.at[pl.ds(pl.multiple_of(r0 + (j + 1) * RCHUNK, RCHUNK),
                                   RCHUNK), :],
                    nbuf, nsem)

            pltpu.make_async_copy(
                x_hbm.at[pl.ds(0, RCHUNK), :], bufs[par], sems[par]).wait()
            buf = bufs[par]

            def cg_body(cg, _, buf=buf):
                col = cg * L
                m1 = state[pl.ds(col, L)]
                m2 = state[pl.ds(C + col, L)]

                def row_body(r, carry):
                    m1, m2 = carry
                    for u in range(RUNROLL):
                        v = buf[r * RUNROLL + u, pl.ds(col, L)]
                        m2 = jnp.maximum(m2, jnp.minimum(m1, v))
                        m1 = jnp.maximum(m1, v)
                    return m1, m2

                m1, m2 = lax.fori_loop(0, RCHUNK // RUNROLL, row_body, (m1, m2))
                state[pl.ds(col, L)] = m1
                state[pl.ds(C + col, L)] = m2
                return 0

            lax.fori_loop(0, CG, cg_body, 0)
        return 0

    lax.fori_loop(0, NCHUNK // 2, chunk_pair, 0)

    pltpu.sync_copy(state, shared.at[pl.ds(pl.multiple_of(sid * 2 * C, 8), 2 * C)])
    plsc.subcore_barrier()
    pltpu.sync_copy(
        shared.at[pl.ds(pl.multiple_of(grp * SLABS * 2 * C, 8), SLABS * 2 * C)], mbuf)

    col0 = slab * CPB

    def mg_body(j, _):
        cc = col0 + j * L
        m1 = jnp.full((L,), _NEG, jnp.float32)
        m2 = jnp.full((L,), _NEG, jnp.float32)
        for t in range(SLABS):
            a1 = mbuf[pl.ds(t * 2 * C + cc, L)]
            a2 = mbuf[pl.ds(t * 2 * C + C + cc, L)]
            m2 = jnp.maximum(jnp.maximum(m2, a2), jnp.minimum(m1, a1))
            m1 = jnp.maximum(m1, a1)
        obuf1[pl.ds(j * L, L)] = m1
        obuf2[pl.ds(j * L, L)] = m2
        return 0

    lax.fori_loop(0, CPB // L, mg_body, 0)
    o_off = pl.multiple_of(b * C + col0, 32)
    pltpu.sync_copy(obuf1, out1_hbm.at[pl.ds(o_off, CPB)])
    pltpu.sync_copy(obuf2, out2_hbm.at[pl.ds(o_off, CPB)])


def _tc_body(x_ref, o1_ref, o2_ref, m1_s, m2_s):
    t = pl.program_id(1)
    nt = pl.num_programs(1)

    @pl.when(t == 0)
    def _():
        m1_s[...] = jnp.full(m1_s.shape, _NEG, jnp.float32)
        m2_s[...] = jnp.full(m2_s.shape, _NEG, jnp.float32)

    v = x_ref[0]
    a, bb = v[:BS // 2], v[BS // 2:]
    m1 = jnp.maximum(a, bb)
    m2 = jnp.minimum(a, bb)
    h = BS // 4
    while h >= 8:
        a1, b1 = m1[:h], m1[h:]
        a2, b2 = m2[:h], m2[h:]
        m2 = jnp.maximum(jnp.maximum(a2, b2), jnp.minimum(a1, b1))
        m1 = jnp.maximum(a1, b1)
        h //= 2
    s1, s2 = m1_s[...], m2_s[...]
    m1_s[...] = jnp.maximum(s1, m1)
    m2_s[...] = jnp.maximum(jnp.maximum(s2, m2), jnp.minimum(s1, m1))

    @pl.when(t == nt - 1)
    def _():
        o1_ref[...] = m1_s[...]
        o2_ref[...] = m2_s[...]


def _merge_body(s1_ref, s2_ref, t1_ref, t2_ref, o1_ref, o2_ref):
    for b in range(B):
        m1 = t1_ref[b * 8:(b + 1) * 8, :]
        m2 = t2_ref[b * 8:(b + 1) * 8, :]
        for h in (4, 2, 1):
            a1, b1 = m1[:h], m1[h:2 * h]
            a2, b2 = m2[:h], m2[h:2 * h]
            m2 = jnp.maximum(jnp.maximum(a2, b2), jnp.minimum(a1, b1))
            m1 = jnp.maximum(a1, b1)
        b1, b2 = m1.reshape(C), m2.reshape(C)
        a1 = s1_ref[pl.ds(b * C, C)]
        a2 = s2_ref[pl.ds(b * C, C)]
        o2_ref[pl.ds(b * C, C)] = jnp.maximum(
            jnp.maximum(a2, b2), jnp.minimum(a1, b1))
        o1_ref[pl.ds(b * C, C)] = jnp.maximum(a1, b1)


def kernel(inputs):
    x2d = inputs.reshape(B * S, C)
    mesh = plsc.VectorSubcoreMesh(
        core_axis_name="c", subcore_axis_name="s", num_cores=2, num_subcores=16)
    sc_k = pl.kernel(
        _sc_body,
        out_type=(jax.ShapeDtypeStruct((B * C,), jnp.float32),
                  jax.ShapeDtypeStruct((B * C,), jnp.float32)),
        mesh=mesh,
        scratch_types=[
            pltpu.VMEM((RCHUNK, C), jnp.float32),
            pltpu.VMEM((RCHUNK, C), jnp.float32),
            pltpu.VMEM((2 * C,), jnp.float32),
            pltpu.VMEM_SHARED((16 * 2 * C,), jnp.float32),
            pltpu.VMEM((SLABS * 2 * C,), jnp.float32),
            pltpu.VMEM((CPB,), jnp.float32),
            pltpu.VMEM((CPB,), jnp.float32),
            pltpu.SemaphoreType.DMA,
            pltpu.SemaphoreType.DMA,
        ],
    )
    sc1, sc2 = sc_k(x2d)

    tc1, tc2 = pl.pallas_call(
        _tc_body,
        grid=(B, TC_ROWS // BS),
        in_specs=[pl.BlockSpec((1, BS, C), lambda b, t: (b, t, 0))],
        out_specs=(pl.BlockSpec((8, C), lambda b, t: (b, 0)),
                   pl.BlockSpec((8, C), lambda b, t: (b, 0))),
        out_shape=(jax.ShapeDtypeStruct((B * 8, C), jnp.float32),
                   jax.ShapeDtypeStruct((B * 8, C), jnp.float32)),
        scratch_shapes=[pltpu.VMEM((8, C), jnp.float32),
                        pltpu.VMEM((8, C), jnp.float32)],
        compiler_params=pltpu.CompilerParams(
            dimension_semantics=("arbitrary", "arbitrary")),
    )(inputs)

    o1, o2 = pl.pallas_call(
        _merge_body,
        out_shape=(jax.ShapeDtypeStruct((B * C,), jnp.float32),
                   jax.ShapeDtypeStruct((B * C,), jnp.float32)),
    )(sc1, sc2, tc1, tc2)

    return jnp.stack([o1, o2], axis=-1).reshape(B, C * K)
